# baseline (speedup 1.0000x reference)
import jax
import jax.numpy as jnp
from jax import lax
from jax.experimental import pallas as pl
from jax.experimental.pallas import tpu as pltpu
from jax.experimental.pallas import tpu_sc as plsc

BATCH = 16384
EMB = 32
N_USERS = 1000000
N_TOURS = 100000

_info = plsc.get_sparse_core_info()
_NC = _info.num_cores
_NS = _info.num_subcores
_L = _info.num_lanes
_NW = _NC * _NS
_BPW = BATCH // _NW

_UCHUNK = 131072
_TGRID_CH = 16384


def _score_fused_body(uT_ref, tT_ref, wu_ref, wt_ref, b_ref,
                      uout_ref, tout_ref):
    uout_ref[...] = jnp.sum(uT_ref[...] * wu_ref[...], axis=0)
    tout_ref[...] = jnp.sum(tT_ref[...] * wt_ref[...], axis=0) + b_ref[0]


def _scores_fused(uT, tT, wu, wt, bias):
    grid = (N_USERS + _UCHUNK - 1) // _UCHUNK
    tlast = (N_TOURS + _TGRID_CH - 1) // _TGRID_CH - 1
    return pl.pallas_call(
        _score_fused_body,
        grid=(grid,),
        in_specs=[
            pl.BlockSpec((EMB, _UCHUNK), lambda i: (0, i)),
            pl.BlockSpec((EMB, _TGRID_CH), lambda i: (0, jnp.minimum(i, tlast))),
            pl.BlockSpec((EMB, 1), lambda i: (0, 0)),
            pl.BlockSpec((EMB, 1), lambda i: (0, 0)),
            pl.BlockSpec(memory_space=pltpu.SMEM),
        ],
        out_specs=[
            pl.BlockSpec((_UCHUNK,), lambda i: (i,)),
            pl.BlockSpec((_TGRID_CH,), lambda i: (jnp.minimum(i, tlast),)),
        ],
        out_shape=[jax.ShapeDtypeStruct((N_USERS,), jnp.float32),
                   jax.ShapeDtypeStruct((N_TOURS,), jnp.float32)],
    )(uT, tT, wu, wt, bias)


def _gather_body(users_hbm, tours_hbm, us_hbm, ts_hbm, out_hbm,
                 uidx, tidx, uval, tval, outv, sem_u, sem_t):
    wid = lax.axis_index("s") * _NC + lax.axis_index("c")
    base = wid * _BPW
    pltpu.sync_copy(users_hbm.at[pl.ds(base, _BPW)], uidx)
    pltpu.sync_copy(tours_hbm.at[pl.ds(base, _BPW)], tidx)
    cu = pltpu.async_copy(us_hbm.at[uidx], uval, sem_u)
    ct = pltpu.async_copy(ts_hbm.at[tidx], tval, sem_t)
    cu.wait()
    ct.wait()

    def group(g, carry):
        sl = pl.ds(g * _L, _L)
        outv[sl] = uval[sl] + tval[sl]
        return carry

    lax.fori_loop(0, _BPW // _L, group, 0)
    pltpu.sync_copy(outv, out_hbm.at[pl.ds(base, _BPW)])


@jax.jit
def kernel(users, tours, user_table, tour_table, W, b):
    wu = W[0, :EMB].reshape(EMB, 1)
    wt = W[0, EMB:].reshape(EMB, 1)
    uscore, tscore = _scores_fused(user_table.T, tour_table.T, wu, wt, b)

    run = pl.kernel(
        _gather_body,
        out_type=jax.ShapeDtypeStruct((BATCH,), jnp.float32),
        mesh=plsc.VectorSubcoreMesh(core_axis_name="c", subcore_axis_name="s"),
        compiler_params=pltpu.CompilerParams(
            needs_layout_passes=False, use_tc_tiling_on_sc=False),
        scratch_types=[
            pltpu.VMEM((_BPW,), jnp.int32),
            pltpu.VMEM((_BPW,), jnp.int32),
            pltpu.VMEM((_BPW,), jnp.float32),
            pltpu.VMEM((_BPW,), jnp.float32),
            pltpu.VMEM((_BPW,), jnp.float32),
            pltpu.SemaphoreType.DMA,
            pltpu.SemaphoreType.DMA,
        ],
    )
    out = run(users.astype(jnp.int32), tours.astype(jnp.int32), uscore, tscore)
    return out.reshape(BATCH, 1)
